# baseline (speedup 1.0000x reference)
import functools

import jax
import jax.numpy as jnp
from jax import lax
from jax.experimental import pallas as pl
from jax.experimental.pallas import tpu as pltpu
from jax.experimental.pallas import tpu_sc as plsc

N_NODES = 10000
D = 128
N_EDGES = 320000

NC = 2
NS = 16
NW = NC * NS

E_CHUNK = 128
CHUNKS_PER_W = 80
E_PER_W = E_CHUNK * CHUNKS_PER_W
E_PAD = E_PER_W * NW

ACC_ROWS = 10112
ROWS_PER_TILE = ACC_ROWS // NS


def _mesh():
    return plsc.VectorSubcoreMesh(core_axis_name="c", subcore_axis_name="s")


def _deg_body(dst_hbm, out_hbm, dst_v, deg_v):
    c = lax.axis_index("c")
    s = lax.axis_index("s")
    w = s * NC + c
    zeros16 = jnp.zeros((16,), jnp.float32)
    ones16 = jnp.ones((16,), jnp.float32)

    def zero_body(i, _):
        deg_v[pl.ds(i * 16, 16)] = zeros16
        return 0

    lax.fori_loop(0, ACC_ROWS // 16, zero_body, 0)
    pltpu.sync_copy(dst_hbm.at[w], dst_v)

    def cnt_body(i, _):
        idx = dst_v[pl.ds(i * 16, 16)]
        plsc.addupdate_scatter(deg_v, [idx], ones16)
        return 0

    lax.fori_loop(0, E_PER_W // 16, cnt_body, 0)
    pltpu.sync_copy(deg_v, out_hbm.at[w])


_deg_kernel = functools.partial(
    pl.kernel,
    out_type=jax.ShapeDtypeStruct((NW, ACC_ROWS), jnp.float32),
    mesh=_mesh(),
    scratch_types=[
        pltpu.VMEM((E_PER_W,), jnp.int32),
        pltpu.VMEM((ACC_ROWS,), jnp.float32),
    ],
    compiler_params=pltpu.CompilerParams(needs_layout_passes=False),
)(_deg_body)


NBUF = 2
SD = 4


def _agg_body(hs_hbm, idx_hbm, zeros_hbm, out_hbm,
              idx_ring, rows_v, acc_sh, gsems, ssems, isems):
    c = lax.axis_index("c")
    s = lax.axis_index("s")
    w = s * NC + c

    pltpu.sync_copy(zeros_hbm, acc_sh.at[pl.ds(s * ROWS_PER_TILE, ROWS_PER_TILE)])
    plsc.subcore_barrier()

    def fetch_idx(chunk, slot):
        pltpu.make_async_copy(idx_hbm.at[w, chunk], idx_ring.at[slot],
                              isems.at[slot]).start()

    def wait_idx(chunk, slot):
        pltpu.make_async_copy(idx_hbm.at[w, chunk], idx_ring.at[slot],
                              isems.at[slot]).wait()

    GSUB = 4
    GS = E_CHUNK // GSUB

    def start_gather(chunk, b, slot):
        for g in range(GSUB):
            pltpu.make_async_copy(
                hs_hbm.at[idx_ring.at[slot, 0, pl.ds(g * GS, GS)]],
                rows_v.at[b, pl.ds(g * GS, GS)], gsems.at[b]).start()

    def wait_gather(b, slot):
        pltpu.make_async_copy(hs_hbm.at[idx_ring.at[slot, 0]], rows_v.at[b],
                              gsems.at[b]).wait()

    def start_scatter(b, slot):
        pltpu.make_async_copy(rows_v.at[b], acc_sh.at[idx_ring.at[slot, 1]],
                              ssems.at[b]).start(add=True)

    def wait_scatter(b, slot):
        pltpu.make_async_copy(rows_v.at[b], acc_sh.at[idx_ring.at[slot, 1]],
                              ssems.at[b]).wait()

    for k in range(SD):
        fetch_idx(k, k)
    wait_idx(0, 0)
    start_gather(0, 0, 0)

    def body(cc, _):
        for r in range(SD):
            chunk = cc * SD + r
            b = r % NBUF
            slot = r

            wait_gather(b, slot)
            start_scatter(b, slot)

            @pl.when(chunk >= 1)
            def _():
                wait_scatter((b + 1) % NBUF, (slot + SD - 1) % SD)

            @pl.when(jnp.logical_and(chunk >= 1, chunk + 3 < CHUNKS_PER_W))
            def _():
                fetch_idx(chunk + 3, (slot + 3) % SD)

            @pl.when(chunk + 1 < CHUNKS_PER_W)
            def _():
                wait_idx(chunk + 1, (slot + 1) % SD)
                start_gather(chunk + 1, (b + 1) % NBUF, (slot + 1) % SD)
        return 0

    lax.fori_loop(0, CHUNKS_PER_W // SD, body, 0, unroll=False)
    wait_scatter((CHUNKS_PER_W - 1) % NBUF, (CHUNKS_PER_W - 1) % SD)
    plsc.subcore_barrier()
    pltpu.sync_copy(acc_sh.at[pl.ds(s * ROWS_PER_TILE, ROWS_PER_TILE)],
                    out_hbm.at[c, pl.ds(s * ROWS_PER_TILE, ROWS_PER_TILE)])


_agg_kernel = functools.partial(
    pl.kernel,
    out_type=jax.ShapeDtypeStruct((NC, ACC_ROWS, D), jnp.float32),
    mesh=_mesh(),
    scratch_types=[
        pltpu.VMEM((SD, 2, E_CHUNK), jnp.int32),
        pltpu.VMEM((NBUF, E_CHUNK, D), jnp.float32),
        pltpu.VMEM_SHARED((ACC_ROWS, D), jnp.float32),
        pltpu.SemaphoreType.DMA((NBUF,)),
        pltpu.SemaphoreType.DMA((NBUF,)),
        pltpu.SemaphoreType.DMA((SD,)),
    ],
)(_agg_body)


BLK = 200


def _dis(degp_blk):
    deg = jnp.sum(degp_blk, axis=0) + 1.0
    return lax.rsqrt(deg)


def _mm_scale_body(x_ref, w_ref, degp_ref, o_ref):
    h = jax.lax.dot_general(x_ref[...], w_ref[...],
                            (((1,), (0,)), ((), ())),
                            preferred_element_type=jnp.float32)
    o_ref[...] = h * _dis(degp_ref[...])


def _mm_scale(x, w, degp):
    return pl.pallas_call(
        _mm_scale_body,
        grid=(N_NODES // BLK,),
        in_specs=[
            pl.BlockSpec((BLK, D), lambda i: (i, 0)),
            pl.BlockSpec((D, D), lambda i: (0, 0)),
            pl.BlockSpec((NW, BLK, 1), lambda i: (0, i, 0)),
        ],
        out_specs=pl.BlockSpec((BLK, D), lambda i: (i, 0)),
        out_shape=jax.ShapeDtypeStruct((N_NODES, D), jnp.float32),
    )(x, w, degp)


def _mid_body(p0_ref, p1_ref, hs_ref, degp_ref, b_ref, w_ref, o_ref):
    dis = _dis(degp_ref[...])
    t = (p0_ref[...] + p1_ref[...] + hs_ref[...]) * dis + b_ref[...]
    t = jnp.maximum(t, 0.0)
    h = jax.lax.dot_general(t, w_ref[...], (((1,), (0,)), ((), ())),
                            preferred_element_type=jnp.float32)
    o_ref[...] = h * dis


def _mid(p0, p1, hs, degp, b, w):
    return pl.pallas_call(
        _mid_body,
        grid=(N_NODES // BLK,),
        in_specs=[
            pl.BlockSpec((BLK, D), lambda i: (i, 0)),
            pl.BlockSpec((BLK, D), lambda i: (i, 0)),
            pl.BlockSpec((BLK, D), lambda i: (i, 0)),
            pl.BlockSpec((NW, BLK, 1), lambda i: (0, i, 0)),
            pl.BlockSpec((1, D), lambda i: (0, 0)),
            pl.BlockSpec((D, D), lambda i: (0, 0)),
        ],
        out_specs=pl.BlockSpec((BLK, D), lambda i: (i, 0)),
        out_shape=jax.ShapeDtypeStruct((N_NODES, D), jnp.float32),
    )(p0, p1, hs, degp, b, w)


def _final_body(p0_ref, p1_ref, hs_ref, degp_ref, b_ref, o_ref):
    dis = _dis(degp_ref[...])
    o_ref[...] = (p0_ref[...] + p1_ref[...] + hs_ref[...]) * dis + b_ref[...]


def _final(p0, p1, hs, degp, b):
    return pl.pallas_call(
        _final_body,
        grid=(N_NODES // BLK,),
        in_specs=[
            pl.BlockSpec((BLK, D), lambda i: (i, 0)),
            pl.BlockSpec((BLK, D), lambda i: (i, 0)),
            pl.BlockSpec((BLK, D), lambda i: (i, 0)),
            pl.BlockSpec((NW, BLK, 1), lambda i: (0, i, 0)),
            pl.BlockSpec((1, D), lambda i: (0, 0)),
        ],
        out_specs=pl.BlockSpec((BLK, D), lambda i: (i, 0)),
        out_shape=jax.ShapeDtypeStruct((N_NODES, D), jnp.float32),
    )(p0, p1, hs, degp, b)


def kernel(x, edge_index, W1, b1, W2, b2):
    src = edge_index[0].astype(jnp.int32)
    dst = edge_index[1].astype(jnp.int32)
    pad = E_PAD - N_EDGES
    srcp = jnp.concatenate([src, jnp.zeros((pad,), jnp.int32)])
    dstp = jnp.concatenate([dst, jnp.full((pad,), N_NODES, jnp.int32)])
    src3 = srcp.reshape(NW, CHUNKS_PER_W, E_CHUNK)
    dst3 = dstp.reshape(NW, CHUNKS_PER_W, E_CHUNK)
    idx4 = jnp.stack([src3, dst3], axis=2)
    dst2 = dstp.reshape(NW, E_PER_W)
    zeros = jnp.zeros((ROWS_PER_TILE, D), jnp.float32)

    degp = _deg_kernel(dst2)
    degp = degp.reshape(NW, ACC_ROWS, 1)[:, :N_NODES, :]

    hs1 = _mm_scale(x, W1, degp)
    p = _agg_kernel(hs1, idx4, zeros)
    hs2 = _mid(p[0, :N_NODES], p[1, :N_NODES], hs1, degp,
               b1.reshape(1, D), W2)
    q = _agg_kernel(hs2, idx4, zeros)
    out = _final(q[0, :N_NODES], q[1, :N_NODES], hs2, degp,
                 b2.reshape(1, D))
    return out


# speedup vs baseline: 2.2695x; 2.2695x over previous
import functools

import jax
import jax.numpy as jnp
from jax import lax
from jax.experimental import pallas as pl
from jax.experimental.pallas import tpu as pltpu
from jax.experimental.pallas import tpu_sc as plsc

N_NODES = 10000
D = 128
N_EDGES = 320000

NC = 2
NS = 16
NW = NC * NS

E_CHUNK = 128
CHUNKS_PER_W = 80
E_PER_W = E_CHUNK * CHUNKS_PER_W
E_PAD = E_PER_W * NW

ACC_ROWS = 10112
ROWS_PER_TILE = ACC_ROWS // NS


def _mesh():
    return plsc.VectorSubcoreMesh(core_axis_name="c", subcore_axis_name="s")


def _deg_body(dst_hbm, out_hbm, dst_v, deg_v):
    c = lax.axis_index("c")
    s = lax.axis_index("s")
    w = s * NC + c
    zeros16 = jnp.zeros((16,), jnp.float32)
    ones16 = jnp.ones((16,), jnp.float32)

    def zero_body(i, _):
        deg_v[pl.ds(i * 16, 16)] = zeros16
        return 0

    lax.fori_loop(0, ACC_ROWS // 16, zero_body, 0)
    pltpu.sync_copy(dst_hbm.at[w], dst_v)

    def cnt_body(i, _):
        idx = dst_v[pl.ds(i * 16, 16)]
        plsc.addupdate_scatter(deg_v, [idx], ones16)
        return 0

    lax.fori_loop(0, E_PER_W // 16, cnt_body, 0)
    pltpu.sync_copy(deg_v, out_hbm.at[w])


_deg_kernel = functools.partial(
    pl.kernel,
    out_type=jax.ShapeDtypeStruct((NW, ACC_ROWS), jnp.float32),
    mesh=_mesh(),
    scratch_types=[
        pltpu.VMEM((E_PER_W,), jnp.int32),
        pltpu.VMEM((ACC_ROWS,), jnp.float32),
    ],
    compiler_params=pltpu.CompilerParams(needs_layout_passes=False),
)(_deg_body)


NBUF = 2
SD = 4


def _agg_body(hs_hbm, idx_hbm, zeros_hbm, out_hbm,
              idx_ring, rows_v, acc_sh, gsems, ssems, isems):
    c = lax.axis_index("c")
    s = lax.axis_index("s")
    w = s * NC + c

    pltpu.sync_copy(zeros_hbm, acc_sh.at[pl.ds(s * ROWS_PER_TILE, ROWS_PER_TILE)])
    plsc.subcore_barrier()

    def fetch_idx(chunk, slot):
        pltpu.make_async_copy(idx_hbm.at[w, chunk], idx_ring.at[slot],
                              isems.at[slot]).start()

    def wait_idx(chunk, slot):
        pltpu.make_async_copy(idx_hbm.at[w, chunk], idx_ring.at[slot],
                              isems.at[slot]).wait()

    GSUB = 4
    GS = E_CHUNK // GSUB

    def start_gather(chunk, b, slot):
        for g in range(GSUB):
            pltpu.make_async_copy(
                hs_hbm.at[idx_ring.at[slot, 0, pl.ds(g * GS, GS)]],
                rows_v.at[b, pl.ds(g * GS, GS)], gsems.at[b]).start()

    def wait_gather(b, slot):
        pltpu.make_async_copy(hs_hbm.at[idx_ring.at[slot, 0]], rows_v.at[b],
                              gsems.at[b]).wait()

    def start_scatter(b, slot):
        pltpu.make_async_copy(rows_v.at[b], acc_sh.at[idx_ring.at[slot, 1]],
                              ssems.at[b]).start(add=True)

    def wait_scatter(b, slot):
        pltpu.make_async_copy(rows_v.at[b], acc_sh.at[idx_ring.at[slot, 1]],
                              ssems.at[b]).wait()

    for k in range(SD):
        fetch_idx(k, k)
    wait_idx(0, 0)
    start_gather(0, 0, 0)

    def body(cc, _):
        for r in range(SD):
            chunk = cc * SD + r
            b = r % NBUF
            slot = r

            wait_gather(b, slot)
            start_scatter(b, slot)

            @pl.when(chunk >= 1)
            def _():
                wait_scatter((b + 1) % NBUF, (slot + SD - 1) % SD)

            @pl.when(jnp.logical_and(chunk >= 1, chunk + 3 < CHUNKS_PER_W))
            def _():
                fetch_idx(chunk + 3, (slot + 3) % SD)

            @pl.when(chunk + 1 < CHUNKS_PER_W)
            def _():
                wait_idx(chunk + 1, (slot + 1) % SD)
                start_gather(chunk + 1, (b + 1) % NBUF, (slot + 1) % SD)
        return 0

    lax.fori_loop(0, CHUNKS_PER_W // SD, body, 0, unroll=False)
    wait_scatter((CHUNKS_PER_W - 1) % NBUF, (CHUNKS_PER_W - 1) % SD)
    plsc.subcore_barrier()
    pltpu.sync_copy(acc_sh.at[pl.ds(s * ROWS_PER_TILE, ROWS_PER_TILE)],
                    out_hbm.at[c, pl.ds(s * ROWS_PER_TILE, ROWS_PER_TILE)])


_agg_kernel = functools.partial(
    pl.kernel,
    out_type=jax.ShapeDtypeStruct((NC, ACC_ROWS, D), jnp.float32),
    mesh=_mesh(),
    scratch_types=[
        pltpu.VMEM((SD, 2, E_CHUNK), jnp.int32),
        pltpu.VMEM((NBUF, E_CHUNK, D), jnp.float32),
        pltpu.VMEM_SHARED((ACC_ROWS, D), jnp.float32),
        pltpu.SemaphoreType.DMA((NBUF,)),
        pltpu.SemaphoreType.DMA((NBUF,)),
        pltpu.SemaphoreType.DMA((SD,)),
    ],
)(_agg_body)


BLK = 200


def _dis(degp_blk):
    deg = jnp.sum(degp_blk, axis=0) + 1.0
    return lax.rsqrt(deg)


def _mm_scale_body(x_ref, w_ref, degp_ref, o_ref):
    h = jax.lax.dot_general(x_ref[...], w_ref[...],
                            (((1,), (0,)), ((), ())),
                            preferred_element_type=jnp.float32)
    o_ref[...] = h * _dis(degp_ref[...])


def _mm_scale(x, w, degp):
    return pl.pallas_call(
        _mm_scale_body,
        grid=(N_NODES // BLK,),
        in_specs=[
            pl.BlockSpec((BLK, D), lambda i: (i, 0)),
            pl.BlockSpec((D, D), lambda i: (0, 0)),
            pl.BlockSpec((NW, BLK, 1), lambda i: (0, i, 0)),
        ],
        out_specs=pl.BlockSpec((BLK, D), lambda i: (i, 0)),
        out_shape=jax.ShapeDtypeStruct((N_NODES, D), jnp.float32),
    )(x, w, degp)


def _mid_body(p0_ref, p1_ref, hs_ref, degp_ref, b_ref, w_ref, o_ref):
    dis = _dis(degp_ref[...])
    t = (p0_ref[...] + p1_ref[...] + hs_ref[...]) * dis + b_ref[...]
    t = jnp.maximum(t, 0.0)
    h = jax.lax.dot_general(t, w_ref[...], (((1,), (0,)), ((), ())),
                            preferred_element_type=jnp.float32)
    o_ref[...] = h * dis


def _mid(p0, p1, hs, degp, b, w):
    return pl.pallas_call(
        _mid_body,
        grid=(N_NODES // BLK,),
        in_specs=[
            pl.BlockSpec((BLK, D), lambda i: (i, 0)),
            pl.BlockSpec((BLK, D), lambda i: (i, 0)),
            pl.BlockSpec((BLK, D), lambda i: (i, 0)),
            pl.BlockSpec((NW, BLK, 1), lambda i: (0, i, 0)),
            pl.BlockSpec((1, D), lambda i: (0, 0)),
            pl.BlockSpec((D, D), lambda i: (0, 0)),
        ],
        out_specs=pl.BlockSpec((BLK, D), lambda i: (i, 0)),
        out_shape=jax.ShapeDtypeStruct((N_NODES, D), jnp.float32),
    )(p0, p1, hs, degp, b, w)


def _final_body(p0_ref, p1_ref, hs_ref, degp_ref, b_ref, o_ref):
    dis = _dis(degp_ref[...])
    o_ref[...] = (p0_ref[...] + p1_ref[...] + hs_ref[...]) * dis + b_ref[...]


def _final(p0, p1, hs, degp, b):
    return pl.pallas_call(
        _final_body,
        grid=(N_NODES // BLK,),
        in_specs=[
            pl.BlockSpec((BLK, D), lambda i: (i, 0)),
            pl.BlockSpec((BLK, D), lambda i: (i, 0)),
            pl.BlockSpec((BLK, D), lambda i: (i, 0)),
            pl.BlockSpec((NW, BLK, 1), lambda i: (0, i, 0)),
            pl.BlockSpec((1, D), lambda i: (0, 0)),
        ],
        out_specs=pl.BlockSpec((BLK, D), lambda i: (i, 0)),
        out_shape=jax.ShapeDtypeStruct((N_NODES, D), jnp.float32),
    )(p0, p1, hs, degp, b)


def kernel(x, edge_index, W1, b1, W2, b2):
    src = edge_index[0].astype(jnp.int32)
    dst = edge_index[1].astype(jnp.int32)
    pad = E_PAD - N_EDGES
    pad_ar = jnp.arange(pad, dtype=jnp.int32)
    srcp = jnp.concatenate([src, pad_ar % N_NODES])
    dstp = jnp.concatenate([dst, N_NODES + pad_ar % (ACC_ROWS - N_NODES)])
    src3 = srcp.reshape(NW, CHUNKS_PER_W, E_CHUNK)
    dst3 = dstp.reshape(NW, CHUNKS_PER_W, E_CHUNK)
    idx4 = jnp.stack([src3, dst3], axis=2)
    dst2 = dstp.reshape(NW, E_PER_W)
    zeros = jnp.zeros((ROWS_PER_TILE, D), jnp.float32)

    degp = _deg_kernel(dst2)
    degp = degp.reshape(NW, ACC_ROWS, 1)[:, :N_NODES, :]

    hs1 = _mm_scale(x, W1, degp)
    p = _agg_kernel(hs1, idx4, zeros)
    hs2 = _mid(p[0, :N_NODES], p[1, :N_NODES], hs1, degp,
               b1.reshape(1, D), W2)
    q = _agg_kernel(hs2, idx4, zeros)
    out = _final(q[0, :N_NODES], q[1, :N_NODES], hs2, degp,
                 b2.reshape(1, D))
    return out
